# baseline (speedup 1.0000x reference)
import functools

import jax
import jax.numpy as jnp
from jax import lax
from jax.experimental import pallas as pl
from jax.experimental.pallas import tpu as pltpu
from jax.experimental.pallas import tpu_sc as plsc

_V = 1000000
_D = 64
_B = 4096
_L = 200
_NC = 2
_NS = 16
_NW = _NC * _NS
_BW = 128
_VBLK = 128
_NBLK = _V // _VBLK
_BPW = _NBLK // _NW

_mesh = plsc.VectorSubcoreMesh(core_axis_name="c", subcore_axis_name="s")

_iota16 = None


@functools.partial(
    pl.kernel,
    mesh=_mesh,
    compiler_params=pltpu.CompilerParams(needs_layout_passes=False),
    out_type=jax.ShapeDtypeStruct((_V, 2 * _D), jnp.float32),
    scratch_types=[
        [pltpu.VMEM((_D, _VBLK), jnp.float32) for _ in range(2)],
        [pltpu.VMEM((_VBLK, 129), jnp.float32) for _ in range(2)],
        [pltpu.SemaphoreType.DMA for _ in range(2)],
        [pltpu.SemaphoreType.DMA for _ in range(2)],
    ],
)
def _format_table(wt_hbm, wtail_hbm, w3_hbm, gbufs, obufs, rsems, wsems):
    wid = lax.axis_index("s") * _NC + lax.axis_index("c")
    base = wid * _BPW

    iota16 = lax.iota(jnp.int32, 16)
    crows = [iota16 + 16 * cg for cg in range(8)]

    def read_cp(g, p):
        return pltpu.make_async_copy(
            wt_hbm.at[:, pl.ds(g * _VBLK, _VBLK)], gbufs[p], rsems[p]
        )

    def write_cp(g, p):
        return pltpu.make_async_copy(
            obufs[p].at[:, pl.ds(0, 2 * _D)],
            w3_hbm.at[pl.ds(g * _VBLK, _VBLK)],
            wsems[p],
        )

    def transpose(p, ncg):
        @pl.loop(0, _D, unroll=4)
        def _per_d(d):
            db = jnp.full((16,), 0, jnp.int32) + d
            for cg in range(ncg):
                vals = gbufs[p][d, pl.ds(16 * cg, 16)]
                plsc.store_scatter(obufs[p], [crows[cg], db], vals)

    read_cp(base, 0).start()
    read_cp(base + 1, 1).start()
    read_cp(base, 0).wait()
    transpose(0, 8)
    write_cp(base, 0).start()
    read_cp(base + 2, 0).start()
    read_cp(base + 1, 1).wait()
    transpose(1, 8)
    write_cp(base + 1, 1).start()
    read_cp(base + 3, 1).start()

    @pl.loop(0, (_BPW - 4) // 2)
    def _steady(i):
        for p in range(2):
            g = base + 2 * i + 2 + p
            read_cp(g, p).wait()
            write_cp(g - 2, p).wait()
            transpose(p, 8)
            write_cp(g, p).start()
            read_cp(g + 2, p).start()

    for p in range(2):
        g = base + _BPW - 2 + p
        read_cp(g, p).wait()
        write_cp(g - 2, p).wait()
        transpose(p, 8)
        write_cp(g, p).start()
    for p in range(2):
        write_cp(base + _BPW - 2 + p, p).wait()

    @pl.when(wid < 4)
    def _tail_full():
        g = _NW * _BPW + wid
        read_cp(g, 0).start()
        read_cp(g, 0).wait()
        transpose(0, 8)
        write_cp(g, 0).start()
        write_cp(g, 0).wait()

    @pl.when(wid == 4)
    def _tail_half():
        v0 = _NW * _BPW * _VBLK + 4 * _VBLK
        pltpu.sync_copy(wtail_hbm, gbufs[0])
        pltpu.sync_copy(gbufs[0], w3_hbm.at[pl.ds(v0, _D)])


@functools.partial(
    pl.kernel,
    mesh=_mesh,
    compiler_params=pltpu.CompilerParams(needs_layout_passes=False),
    out_type=jax.ShapeDtypeStruct((_L, 8, _NW, 8, 128), jnp.float32),
    scratch_types=[
        pltpu.VMEM((_L, _BW), jnp.int32),
        [pltpu.VMEM((_BW, 2 * _D), jnp.float32) for _ in range(2)],
        [pltpu.VMEM((_D, 129), jnp.float32) for _ in range(2)],
        [pltpu.SemaphoreType.DMA for _ in range(2)],
        [pltpu.SemaphoreType.DMA for _ in range(2)],
    ],
)
def _embedding_gather(w_hbm, idx_hbm, out_hbm, idx_v, gbufs, obufs, gsems, osems):
    wid = lax.axis_index("s") * _NC + lax.axis_index("c")

    pltpu.sync_copy(idx_hbm.at[:, pl.ds(wid * _BW, _BW)], idx_v)

    def gather_cp(l, p):
        return pltpu.make_async_copy(w_hbm.at[idx_v.at[l]], gbufs[p], gsems[p])

    def out_cp(l, p, di):
        return pltpu.make_async_copy(
            obufs[p].at[pl.ds(di * 8, 8), pl.ds(0, 128)],
            out_hbm.at[l, di, wid],
            osems[p],
        )

    def out_start(l, p):
        for di in range(8):
            out_cp(l, p, di).start()

    def out_wait(l, p):
        for di in range(8):
            out_cp(l, p, di).wait()

    iota16 = lax.iota(jnp.int32, 16)
    didx = [iota16 + 16 * k for k in range(4)]

    def transpose(p):
        @pl.loop(0, _BW, unroll=8)
        def _per_c(c):
            cb = jnp.full((16,), 0, jnp.int32) + c
            for k in range(4):
                vals = gbufs[p][c, pl.ds(16 * k, 16)]
                plsc.store_scatter(obufs[p], [didx[k], cb], vals)

    gather_cp(0, 0).start()
    gather_cp(1, 1).start()
    gather_cp(0, 0).wait()
    transpose(0)
    out_start(0, 0)
    gather_cp(2, 0).start()
    gather_cp(1, 1).wait()
    transpose(1)
    out_start(1, 1)
    gather_cp(3, 1).start()

    @pl.loop(0, (_L - 4) // 2)
    def _steady(i):
        for p in range(2):
            l = 2 * i + 2 + p
            gather_cp(l, p).wait()
            out_wait(l - 2, p)
            transpose(p)
            out_start(l, p)
            gather_cp(l + 2, p).start()

    for p in range(2):
        l = _L - 2 + p
        gather_cp(l, p).wait()
        out_wait(l - 2, p)
        transpose(p)
        out_start(l, p)
    for p in range(2):
        out_wait(_L - 2 + p, p)


def kernel(x, w):
    wtail = jnp.pad(w[_NBLK * _VBLK:], ((0, 0), (0, _D)))
    w3 = _format_table(w.T, wtail)
    out5 = _embedding_gather(w3, x.T)
    return out5.transpose(2, 4, 0, 1, 3).reshape(_B, _L, _D)


# speedup vs baseline: 2.4027x; 2.4027x over previous
import functools

import jax
import jax.numpy as jnp
from jax import lax
from jax.experimental import pallas as pl
from jax.experimental.pallas import tpu as pltpu
from jax.experimental.pallas import tpu_sc as plsc

_V = 1000000
_D = 64
_B = 4096
_L = 200
_NC = 2
_NS = 16
_NW = _NC * _NS
_BW = 128

_mesh = plsc.VectorSubcoreMesh(core_axis_name="c", subcore_axis_name="s")


@functools.partial(
    pl.kernel,
    mesh=_mesh,
    compiler_params=pltpu.CompilerParams(
        use_tc_tiling_on_sc=False, needs_layout_passes=False
    ),
    out_type=jax.ShapeDtypeStruct((_L, 8, _NW, 8, 128), jnp.float32),
    scratch_types=[
        pltpu.VMEM((_L, _BW), jnp.int32),
        [pltpu.VMEM((_BW, 2 * _D), jnp.float32) for _ in range(2)],
        [pltpu.VMEM((_D, 129), jnp.float32) for _ in range(2)],
        [pltpu.SemaphoreType.DMA for _ in range(2)],
        [pltpu.SemaphoreType.DMA for _ in range(2)],
    ],
)
def _embedding_gather(w_hbm, idx_hbm, out_hbm, idx_v, gbufs, obufs, gsems, osems):
    wid = lax.axis_index("s") * _NC + lax.axis_index("c")

    pltpu.sync_copy(idx_hbm.at[:, pl.ds(wid * _BW, _BW)], idx_v)

    def gather_cp(l, p):
        return pltpu.make_async_copy(w_hbm.at[idx_v.at[l]], gbufs[p], gsems[p])

    def out_cp(l, p, di):
        return pltpu.make_async_copy(
            obufs[p].at[pl.ds(di * 8, 8), pl.ds(0, 128)],
            out_hbm.at[l, di, wid],
            osems[p],
        )

    def out_start(l, p):
        for di in range(8):
            out_cp(l, p, di).start()

    def out_wait(l, p):
        for di in range(8):
            out_cp(l, p, di).wait()

    iota16 = lax.iota(jnp.int32, 16)
    didx = [iota16 + 16 * k for k in range(4)]

    def transpose(p):
        @pl.loop(0, _BW, unroll=8)
        def _per_c(c):
            cb = jnp.full((16,), 0, jnp.int32) + c
            for k in range(4):
                vals = gbufs[p][c, pl.ds(16 * k, 16)]
                plsc.store_scatter(obufs[p], [didx[k], cb], vals)

    gather_cp(0, 0).start()
    gather_cp(1, 1).start()
    gather_cp(0, 0).wait()
    transpose(0)
    out_start(0, 0)
    gather_cp(2, 0).start()
    gather_cp(1, 1).wait()
    transpose(1)
    out_start(1, 1)
    gather_cp(3, 1).start()

    @pl.loop(0, (_L - 4) // 2)
    def _steady(i):
        for p in range(2):
            l = 2 * i + 2 + p
            gather_cp(l, p).wait()
            out_wait(l - 2, p)
            transpose(p)
            out_start(l, p)
            gather_cp(l + 2, p).start()

    for p in range(2):
        l = _L - 2 + p
        gather_cp(l, p).wait()
        out_wait(l - 2, p)
        transpose(p)
        out_start(l, p)
    for p in range(2):
        out_wait(_L - 2 + p, p)


def kernel(x, w):
    wp = jnp.pad(w, ((0, 0), (0, _D)))
    out5 = _embedding_gather(wp, x.T)
    return out5.transpose(2, 4, 0, 1, 3).reshape(_B, _L, _D)
